# baseline (speedup 1.0000x reference)
import functools

import jax
import jax.numpy as jnp
from jax import lax
from jax.experimental import pallas as pl
from jax.experimental.pallas import tpu as pltpu
from jax.experimental.pallas import tpu_sc as plsc

N = 10000
NPAD = 10240
D_IN = 256
F = 128
N_LAYERS = 3
E_REAL = 170000
NW = 32
C = 128
NCH0 = 56
NCH1 = 32
NCHT = 16 * (NCH0 + NCH1)
EPAD = NCHT * C
STRIPE = NPAD // 16


def _elu(x):
    return jnp.where(x > 0, x, jnp.exp(x) - 1.0)


def _leaky(x):
    return jnp.maximum(x, 0.2 * x)


def _att_out(z, asv, adv, a_ref, d_ref, u_ref):
    a_s = jnp.sum(z * asv, axis=1, keepdims=True)
    a_d = jnp.sum(z * adv, axis=1, keepdims=True)
    amax = jnp.max(a_s)
    u = _leaky(amax + a_d)
    pad = jnp.zeros((NPAD - N, 1), jnp.float32)
    a_ref[pl.ds(0, N), :] = a_s
    a_ref[pl.ds(N, NPAD - N), :] = pad
    d_ref[pl.ds(0, N), :] = a_d
    d_ref[pl.ds(N, NPAD - N), :] = pad
    u_ref[pl.ds(0, N), :] = u
    u_ref[pl.ds(N, NPAD - N), :] = pad


def _head_body(x_ref, w_ref, b_ref, g_ref, beta_ref, gw_ref, asv_ref, adv_ref,
               h_ref, z_ref, a_ref, d_ref, u_ref):
    h = jnp.dot(x_ref[...], w_ref[...].T, preferred_element_type=jnp.float32)
    h = h + b_ref[...]
    mu = jnp.mean(h, axis=1, keepdims=True)
    var = jnp.mean((h - mu) ** 2, axis=1, keepdims=True)
    h = (h - mu) / jnp.sqrt(var + 1e-5) * g_ref[...] + beta_ref[...]
    h = _elu(h)
    h_ref[...] = h
    z = jnp.dot(h, gw_ref[...].T, preferred_element_type=jnp.float32)
    z_ref[...] = z
    _att_out(z, asv_ref[...], adv_ref[...], a_ref, d_ref, u_ref)


def _head_call(x, w, b, g, beta, gw, asv, adv):
    return pl.pallas_call(
        _head_body,
        out_shape=(
            jax.ShapeDtypeStruct((N, F), jnp.float32),
            jax.ShapeDtypeStruct((N, F), jnp.float32),
            jax.ShapeDtypeStruct((NPAD, 1), jnp.float32),
            jax.ShapeDtypeStruct((NPAD, 1), jnp.float32),
            jax.ShapeDtypeStruct((NPAD, 1), jnp.float32),
        ),
    )(x, w, b, g, beta, gw, asv, adv)


def _post_h(num_ref, den_ref, bias_ref, bnw_ref, bnb_ref, hin_ref):
    n = num_ref[0, pl.ds(0, N), :] + num_ref[1, pl.ds(0, N), :]
    d = den_ref[0, pl.ds(0, N), :] + den_ref[1, pl.ds(0, N), :]
    out = n / (d + 1e-16) + bias_ref[...]
    out = out / jnp.sqrt(1.0 + 1e-5) * bnw_ref[...] + bnb_ref[...]
    return _elu(out) + hin_ref[...]


def _mid_body(num_ref, den_ref, bias_ref, bnw_ref, bnb_ref, hin_ref,
              gw_ref, asv_ref, adv_ref,
              h_ref, z_ref, a_ref, d_ref, u_ref):
    h = _post_h(num_ref, den_ref, bias_ref, bnw_ref, bnb_ref, hin_ref)
    h_ref[...] = h
    z = jnp.dot(h, gw_ref[...].T, preferred_element_type=jnp.float32)
    z_ref[...] = z
    _att_out(z, asv_ref[...], adv_ref[...], a_ref, d_ref, u_ref)


def _mid_call(num, den, bias, bnw, bnb, hin, gw, asv, adv):
    return pl.pallas_call(
        _mid_body,
        out_shape=(
            jax.ShapeDtypeStruct((N, F), jnp.float32),
            jax.ShapeDtypeStruct((N, F), jnp.float32),
            jax.ShapeDtypeStruct((NPAD, 1), jnp.float32),
            jax.ShapeDtypeStruct((NPAD, 1), jnp.float32),
            jax.ShapeDtypeStruct((NPAD, 1), jnp.float32),
        ),
    )(num, den.reshape(2, NPAD, 1), bias, bnw, bnb, hin, gw, asv, adv)


_SC_MESH = plsc.VectorSubcoreMesh(core_axis_name="c", subcore_axis_name="s")


@functools.partial(
    pl.kernel,
    out_type=(
        jax.ShapeDtypeStruct((2, NPAD, F), jnp.float32),
        jax.ShapeDtypeStruct((2, NPAD), jnp.float32),
    ),
    mesh=_SC_MESH,
    compiler_params=pltpu.CompilerParams(needs_layout_passes=False),
    scratch_types=[
        pltpu.VMEM((NCH0, C), jnp.int32),
        pltpu.VMEM((NCH0, C), jnp.int32),
        pltpu.VMEM((2, C), jnp.float32),
        pltpu.VMEM((2, C), jnp.float32),
        pltpu.VMEM((2, C), jnp.float32),
        pltpu.VMEM((2, C), jnp.float32),
        pltpu.VMEM((2, C, F), jnp.float32),
        pltpu.VMEM_SHARED((NPAD, F), jnp.float32),
        pltpu.VMEM_SHARED((NPAD,), jnp.float32),
        pltpu.SemaphoreType.DMA,
        pltpu.SemaphoreType.DMA,
    ],
)
def _edge_kernel(src_hbm, dst_hbm, asrc_hbm, adst_hbm, u_hbm, z_hbm,
                 zrow_hbm, zden_hbm, num_hbm, den_hbm,
                 src_v, dst_v, as_v, ad_v, uu_v, w_v, rows_v,
                 num_sh, den_sh, sem_g, sem_s):
    c = lax.axis_index("c")
    s = lax.axis_index("s")

    pltpu.sync_copy(zrow_hbm, num_sh.at[pl.ds(s * STRIPE, STRIPE)])
    pltpu.sync_copy(zden_hbm, den_sh.at[pl.ds(s * STRIPE, STRIPE)])
    plsc.subcore_barrier()

    def issue_gather(j, b):
        pltpu.async_copy(z_hbm.at[src_v.at[j]], rows_v.at[b], sem_g)
        pltpu.async_copy(asrc_hbm.at[src_v.at[j]], as_v.at[b], sem_g)
        pltpu.async_copy(adst_hbm.at[dst_v.at[j]], ad_v.at[b], sem_g)
        pltpu.async_copy(u_hbm.at[dst_v.at[j]], uu_v.at[b], sem_g)

    def wait_gather(j, b):
        pltpu.make_async_copy(z_hbm.at[src_v.at[j]], rows_v.at[b], sem_g).wait()
        pltpu.make_async_copy(asrc_hbm.at[src_v.at[j]], as_v.at[b], sem_g).wait()
        pltpu.make_async_copy(adst_hbm.at[dst_v.at[j]], ad_v.at[b], sem_g).wait()
        pltpu.make_async_copy(u_hbm.at[dst_v.at[j]], uu_v.at[b], sem_g).wait()

    def issue_scatter(j, b):
        pltpu.async_copy(w_v.at[b], den_sh.at[dst_v.at[j]], sem_s, add=True)
        pltpu.async_copy(rows_v.at[b], num_sh.at[dst_v.at[j]], sem_s, add=True)

    def wait_scatter(j, b):
        pltpu.make_async_copy(w_v.at[b], den_sh.at[dst_v.at[j]], sem_s).wait()
        pltpu.make_async_copy(rows_v.at[b], num_sh.at[dst_v.at[j]], sem_s).wait()

    def compute(b):
        @pl.loop(0, C // 16)
        def _w_loop(g):
            k = g * 16
            t = _leaky(as_v[b, pl.ds(k, 16)] + ad_v[b, pl.ds(k, 16)])
            w_v[b, pl.ds(k, 16)] = jnp.exp(t - uu_v[b, pl.ds(k, 16)])

        @plsc.parallel_loop(0, C, unroll=4)
        def _scale(e):
            wsp = plsc.load_gather(
                w_v, [jnp.full((16,), b, jnp.int32), jnp.full((16,), e, jnp.int32)])
            for kk in range(F // 16):
                rows_v[b, e, pl.ds(kk * 16, 16)] = (
                    rows_v[b, e, pl.ds(kk * 16, 16)] * wsp)

    def run_chunks(start, nch):
        pltpu.sync_copy(src_hbm.at[pl.ds(start, nch)], src_v.at[pl.ds(0, nch)])
        pltpu.sync_copy(dst_hbm.at[pl.ds(start, nch)], dst_v.at[pl.ds(0, nch)])
        issue_gather(0, 0)

        @pl.loop(0, nch, step=2)
        def _row_loop(j):
            @pl.when(j > 0)
            def _():
                wait_scatter(j - 1, 1)
            issue_gather(j + 1, 1)
            wait_gather(j, 0)
            compute(0)
            issue_scatter(j, 0)
            wait_gather(j + 1, 1)
            wait_scatter(j, 0)
            @pl.when(j + 2 < nch)
            def _():
                issue_gather(j + 2, 0)
            compute(1)
            issue_scatter(j + 1, 1)

        wait_scatter(nch - 1, 1)

    @pl.when(c == 0)
    def _():
        run_chunks(s * NCH0, NCH0)

    @pl.when(c == 1)
    def _():
        run_chunks(16 * NCH0 + s * NCH1, NCH1)

    plsc.subcore_barrier()
    pltpu.sync_copy(num_sh.at[pl.ds(s * STRIPE, STRIPE)],
                    num_hbm.at[c, pl.ds(s * STRIPE, STRIPE)])
    pltpu.sync_copy(den_sh.at[pl.ds(s * STRIPE, STRIPE)],
                    den_hbm.at[c, pl.ds(s * STRIPE, STRIPE)])


def _pool_body(num_ref, den_ref, bias_ref, bnw_ref, bnb_ref, hin_ref,
               mw_ref, mb_ref, xw_ref, xb_ref, a1w_ref, a1b_ref,
               a2w_ref, a2b_ref, f1w_ref, f1b_ref, f2w_ref, f2b_ref,
               fg_ref, fbeta_ref, out_ref):
    h = _post_h(num_ref, den_ref, bias_ref, bnw_ref, bnb_ref, hin_ref)
    mean_h = jnp.mean(h, axis=0, keepdims=True)
    mp = jnp.dot(mean_h, mw_ref[...].T, preferred_element_type=jnp.float32)
    mp = mp + mb_ref[...]
    max_h = jnp.max(h, axis=0, keepdims=True)
    xp = jnp.dot(max_h, xw_ref[...].T, preferred_element_type=jnp.float32)
    xp = xp + xb_ref[...]
    s1 = jnp.dot(h, a1w_ref[...].T, preferred_element_type=jnp.float32)
    s1 = jnp.maximum(s1 + a1b_ref[...], 0.0)
    s = jnp.sum(s1 * a2w_ref[...], axis=1, keepdims=True)
    s = s + a2b_ref[0, 0]
    smax = jnp.max(s)
    es = jnp.exp(s - smax)
    sw = es / jnp.sum(es)
    ah = jnp.sum(h * sw, axis=0, keepdims=True)
    ap = jnp.dot(ah, mw_ref[...].T, preferred_element_type=jnp.float32)
    comb = jnp.concatenate([mp, xp, ap], axis=1)
    f = jnp.dot(comb, f1w_ref[...].T, preferred_element_type=jnp.float32)
    f = jnp.maximum(f + f1b_ref[...], 0.0)
    f = jnp.dot(f, f2w_ref[...].T, preferred_element_type=jnp.float32)
    f = f + f2b_ref[...]
    mu = jnp.mean(f, axis=1, keepdims=True)
    var = jnp.mean((f - mu) ** 2, axis=1, keepdims=True)
    out_ref[...] = (f - mu) / jnp.sqrt(var + 1e-5) * fg_ref[...] + fbeta_ref[...]


def _pool_call(num, den, bias, bnw, bnb, hin, p):
    row = lambda v: v.reshape(1, -1)
    return pl.pallas_call(
        _pool_body,
        out_shape=jax.ShapeDtypeStruct((1, F), jnp.float32),
    )(num, den.reshape(2, NPAD, 1), bias, bnw, bnb, hin,
      p['mean_w'], row(p['mean_b']), p['max_w'], row(p['max_b']),
      p['attn_w1'], row(p['attn_b1']), p['attn_w2'], row(p['attn_b2']),
      p['fus_w1'], row(p['fus_b1']), p['fus_w2'], row(p['fus_b2']),
      row(p['fus_g']), row(p['fus_beta']))


def kernel(x, edge_index, params):
    p = params
    row = lambda v: v.reshape(1, -1)
    loops_idx = jnp.arange(N, dtype=jnp.int32)
    npad_e = EPAD - E_REAL
    src = jnp.concatenate([edge_index[0].astype(jnp.int32), loops_idx,
                           jnp.zeros((npad_e,), jnp.int32)])
    dst = jnp.concatenate([edge_index[1].astype(jnp.int32), loops_idx,
                           jnp.full((npad_e,), N, jnp.int32)])
    src3 = src.reshape(NCHT, C)
    dst3 = dst.reshape(NCHT, C)
    zrow = jnp.zeros((STRIPE, F), jnp.float32)
    zden = jnp.zeros((STRIPE,), jnp.float32)

    g0 = p['gat'][0]
    h, z, a_s, a_d, u = _head_call(
        x, p['proj_w'], row(p['proj_b']), row(p['ln_g']), row(p['ln_b']),
        g0['W'], row(g0['att_src'][0]), row(g0['att_dst'][0]))
    for i in range(N_LAYERS):
        g = p['gat'][i]
        num, den = _edge_kernel(src3, dst3,
                                a_s.reshape(NPAD), a_d.reshape(NPAD),
                                u.reshape(NPAD), z, zrow, zden)
        bias, bnw, bnb = (row(g['bias']), row(p['bn'][i]['w']),
                          row(p['bn'][i]['b']))
        if i < N_LAYERS - 1:
            gn = p['gat'][i + 1]
            h, z, a_s, a_d, u = _mid_call(
                num, den, bias, bnw, bnb, h,
                gn['W'], row(gn['att_src'][0]), row(gn['att_dst'][0]))
        else:
            return _pool_call(num, den, bias, bnw, bnb, h, p)


# speedup vs baseline: 2.5110x; 2.5110x over previous
import functools

import jax
import jax.numpy as jnp
from jax import lax
from jax.experimental import pallas as pl
from jax.experimental.pallas import tpu as pltpu
from jax.experimental.pallas import tpu_sc as plsc

N = 10000
NPAD = 10240
D_IN = 256
F = 128
N_LAYERS = 3
E_REAL = 170000
NW = 32
C = 128
NCH = 42
NCHT = NW * NCH
EPAD = NCHT * C
STRIPE = NPAD // 16


def _elu(x):
    return jnp.where(x > 0, x, jnp.exp(x) - 1.0)


def _leaky(x):
    return jnp.maximum(x, 0.2 * x)


def _att_out(z, asv, adv, a_ref, d_ref, u_ref):
    a_s = jnp.sum(z * asv, axis=1, keepdims=True)
    a_d = jnp.sum(z * adv, axis=1, keepdims=True)
    amax = jnp.max(a_s)
    u = _leaky(amax + a_d)
    pad = jnp.zeros((NPAD - N, 1), jnp.float32)
    a_ref[pl.ds(0, N), :] = a_s
    a_ref[pl.ds(N, NPAD - N), :] = pad
    d_ref[pl.ds(0, N), :] = a_d
    d_ref[pl.ds(N, NPAD - N), :] = pad
    u_ref[pl.ds(0, N), :] = u
    u_ref[pl.ds(N, NPAD - N), :] = pad


def _head_body(x_ref, w_ref, b_ref, g_ref, beta_ref, gw_ref, asv_ref, adv_ref,
               h_ref, z_ref, a_ref, d_ref, u_ref):
    h = jnp.dot(x_ref[...], w_ref[...].T, preferred_element_type=jnp.float32)
    h = h + b_ref[...]
    mu = jnp.mean(h, axis=1, keepdims=True)
    var = jnp.mean((h - mu) ** 2, axis=1, keepdims=True)
    h = (h - mu) / jnp.sqrt(var + 1e-5) * g_ref[...] + beta_ref[...]
    h = _elu(h)
    h_ref[...] = h
    z = jnp.dot(h, gw_ref[...].T, preferred_element_type=jnp.float32)
    z_ref[...] = z
    _att_out(z, asv_ref[...], adv_ref[...], a_ref, d_ref, u_ref)


def _head_call(x, w, b, g, beta, gw, asv, adv):
    return pl.pallas_call(
        _head_body,
        out_shape=(
            jax.ShapeDtypeStruct((N, F), jnp.float32),
            jax.ShapeDtypeStruct((N, F), jnp.float32),
            jax.ShapeDtypeStruct((NPAD, 1), jnp.float32),
            jax.ShapeDtypeStruct((NPAD, 1), jnp.float32),
            jax.ShapeDtypeStruct((NPAD, 1), jnp.float32),
        ),
    )(x, w, b, g, beta, gw, asv, adv)


def _post_h(num_ref, den_ref, bias_ref, bnw_ref, bnb_ref, hin_ref):
    n = num_ref[0, pl.ds(0, N), :] + num_ref[1, pl.ds(0, N), :]
    d = den_ref[0, pl.ds(0, N), :] + den_ref[1, pl.ds(0, N), :]
    out = n / (d + 1e-16) + bias_ref[...]
    out = out / jnp.sqrt(1.0 + 1e-5) * bnw_ref[...] + bnb_ref[...]
    return _elu(out) + hin_ref[...]


def _mid_body(num_ref, den_ref, bias_ref, bnw_ref, bnb_ref, hin_ref,
              gw_ref, asv_ref, adv_ref,
              h_ref, z_ref, a_ref, d_ref, u_ref):
    h = _post_h(num_ref, den_ref, bias_ref, bnw_ref, bnb_ref, hin_ref)
    h_ref[...] = h
    z = jnp.dot(h, gw_ref[...].T, preferred_element_type=jnp.float32)
    z_ref[...] = z
    _att_out(z, asv_ref[...], adv_ref[...], a_ref, d_ref, u_ref)


def _mid_call(num, den, bias, bnw, bnb, hin, gw, asv, adv):
    return pl.pallas_call(
        _mid_body,
        out_shape=(
            jax.ShapeDtypeStruct((N, F), jnp.float32),
            jax.ShapeDtypeStruct((N, F), jnp.float32),
            jax.ShapeDtypeStruct((NPAD, 1), jnp.float32),
            jax.ShapeDtypeStruct((NPAD, 1), jnp.float32),
            jax.ShapeDtypeStruct((NPAD, 1), jnp.float32),
        ),
    )(num, den.reshape(2, NPAD, 1), bias, bnw, bnb, hin, gw, asv, adv)


_SC_MESH = plsc.VectorSubcoreMesh(core_axis_name="c", subcore_axis_name="s")


@functools.partial(
    pl.kernel,
    out_type=(
        jax.ShapeDtypeStruct((2, NPAD, F), jnp.float32),
        jax.ShapeDtypeStruct((2, NPAD), jnp.float32),
    ),
    mesh=_SC_MESH,
    compiler_params=pltpu.CompilerParams(needs_layout_passes=False),
    scratch_types=[
        pltpu.VMEM((NCH, C), jnp.int32),
        pltpu.VMEM((NCH, C), jnp.int32),
        pltpu.VMEM((2, C), jnp.float32),
        pltpu.VMEM((2, C), jnp.float32),
        pltpu.VMEM((2, C), jnp.float32),
        pltpu.VMEM((2, C), jnp.float32),
        pltpu.VMEM((2, C, F), jnp.float32),
        pltpu.VMEM_SHARED((NPAD, F), jnp.float32),
        pltpu.VMEM_SHARED((NPAD,), jnp.float32),
        pltpu.SemaphoreType.DMA,
        pltpu.SemaphoreType.DMA,
    ],
)
def _edge_kernel(src_hbm, dst_hbm, asrc_hbm, adst_hbm, u_hbm, z_hbm,
                 zrow_hbm, zden_hbm, num_hbm, den_hbm,
                 src_v, dst_v, as_v, ad_v, uu_v, w_v, rows_v,
                 num_sh, den_sh, sem_g, sem_s):
    c = lax.axis_index("c")
    s = lax.axis_index("s")
    wid = c * 16 + s

    pltpu.sync_copy(zrow_hbm, num_sh.at[pl.ds(s * STRIPE, STRIPE)])
    pltpu.sync_copy(zden_hbm, den_sh.at[pl.ds(s * STRIPE, STRIPE)])
    pltpu.sync_copy(src_hbm.at[wid], src_v)
    pltpu.sync_copy(dst_hbm.at[wid], dst_v)
    plsc.subcore_barrier()

    def issue_gather(j, b):
        pltpu.async_copy(z_hbm.at[src_v.at[j]], rows_v.at[b], sem_g)
        pltpu.async_copy(asrc_hbm.at[src_v.at[j]], as_v.at[b], sem_g)
        pltpu.async_copy(adst_hbm.at[dst_v.at[j]], ad_v.at[b], sem_g)
        pltpu.async_copy(u_hbm.at[dst_v.at[j]], uu_v.at[b], sem_g)

    def wait_gather(j, b):
        pltpu.make_async_copy(z_hbm.at[src_v.at[j]], rows_v.at[b], sem_g).wait()
        pltpu.make_async_copy(asrc_hbm.at[src_v.at[j]], as_v.at[b], sem_g).wait()
        pltpu.make_async_copy(adst_hbm.at[dst_v.at[j]], ad_v.at[b], sem_g).wait()
        pltpu.make_async_copy(u_hbm.at[dst_v.at[j]], uu_v.at[b], sem_g).wait()

    def issue_scatter(j, b):
        pltpu.async_copy(w_v.at[b], den_sh.at[dst_v.at[j]], sem_s, add=True)
        pltpu.async_copy(rows_v.at[b], num_sh.at[dst_v.at[j]], sem_s, add=True)

    def wait_scatter(j, b):
        pltpu.make_async_copy(w_v.at[b], den_sh.at[dst_v.at[j]], sem_s).wait()
        pltpu.make_async_copy(rows_v.at[b], num_sh.at[dst_v.at[j]], sem_s).wait()

    def compute(b):
        @pl.loop(0, C // 16)
        def _w_loop(g):
            k = g * 16
            t = _leaky(as_v[b, pl.ds(k, 16)] + ad_v[b, pl.ds(k, 16)])
            w_v[b, pl.ds(k, 16)] = jnp.exp(t - uu_v[b, pl.ds(k, 16)])

        @plsc.parallel_loop(0, C, unroll=4)
        def _scale(e):
            wsp = plsc.load_gather(
                w_v, [jnp.full((16,), b, jnp.int32), jnp.full((16,), e, jnp.int32)])
            for kk in range(F // 16):
                rows_v[b, e, pl.ds(kk * 16, 16)] = (
                    rows_v[b, e, pl.ds(kk * 16, 16)] * wsp)

    issue_gather(0, 0)

    @pl.loop(0, NCH, step=2)
    def _row_loop(j):
        @pl.when(j > 0)
        def _():
            wait_scatter(j - 1, 1)
        issue_gather(j + 1, 1)
        wait_gather(j, 0)
        compute(0)
        issue_scatter(j, 0)
        wait_gather(j + 1, 1)
        wait_scatter(j, 0)
        @pl.when(j + 2 < NCH)
        def _():
            issue_gather(j + 2, 0)
        compute(1)
        issue_scatter(j + 1, 1)

    wait_scatter(NCH - 1, 1)

    plsc.subcore_barrier()
    pltpu.sync_copy(num_sh.at[pl.ds(s * STRIPE, STRIPE)],
                    num_hbm.at[c, pl.ds(s * STRIPE, STRIPE)])
    pltpu.sync_copy(den_sh.at[pl.ds(s * STRIPE, STRIPE)],
                    den_hbm.at[c, pl.ds(s * STRIPE, STRIPE)])


def _pool_body(num_ref, den_ref, bias_ref, bnw_ref, bnb_ref, hin_ref,
               mw_ref, mb_ref, xw_ref, xb_ref, a1w_ref, a1b_ref,
               a2w_ref, a2b_ref, f1w_ref, f1b_ref, f2w_ref, f2b_ref,
               fg_ref, fbeta_ref, out_ref):
    h = _post_h(num_ref, den_ref, bias_ref, bnw_ref, bnb_ref, hin_ref)
    mean_h = jnp.mean(h, axis=0, keepdims=True)
    mp = jnp.dot(mean_h, mw_ref[...].T, preferred_element_type=jnp.float32)
    mp = mp + mb_ref[...]
    max_h = jnp.max(h, axis=0, keepdims=True)
    xp = jnp.dot(max_h, xw_ref[...].T, preferred_element_type=jnp.float32)
    xp = xp + xb_ref[...]
    s1 = jnp.dot(h, a1w_ref[...].T, preferred_element_type=jnp.float32)
    s1 = jnp.maximum(s1 + a1b_ref[...], 0.0)
    s = jnp.sum(s1 * a2w_ref[...], axis=1, keepdims=True)
    s = s + a2b_ref[0, 0]
    smax = jnp.max(s)
    es = jnp.exp(s - smax)
    sw = es / jnp.sum(es)
    ah = jnp.sum(h * sw, axis=0, keepdims=True)
    ap = jnp.dot(ah, mw_ref[...].T, preferred_element_type=jnp.float32)
    comb = jnp.concatenate([mp, xp, ap], axis=1)
    f = jnp.dot(comb, f1w_ref[...].T, preferred_element_type=jnp.float32)
    f = jnp.maximum(f + f1b_ref[...], 0.0)
    f = jnp.dot(f, f2w_ref[...].T, preferred_element_type=jnp.float32)
    f = f + f2b_ref[...]
    mu = jnp.mean(f, axis=1, keepdims=True)
    var = jnp.mean((f - mu) ** 2, axis=1, keepdims=True)
    out_ref[...] = (f - mu) / jnp.sqrt(var + 1e-5) * fg_ref[...] + fbeta_ref[...]


def _pool_call(num, den, bias, bnw, bnb, hin, p):
    row = lambda v: v.reshape(1, -1)
    return pl.pallas_call(
        _pool_body,
        out_shape=jax.ShapeDtypeStruct((1, F), jnp.float32),
    )(num, den.reshape(2, NPAD, 1), bias, bnw, bnb, hin,
      p['mean_w'], row(p['mean_b']), p['max_w'], row(p['max_b']),
      p['attn_w1'], row(p['attn_b1']), p['attn_w2'], row(p['attn_b2']),
      p['fus_w1'], row(p['fus_b1']), p['fus_w2'], row(p['fus_b2']),
      row(p['fus_g']), row(p['fus_beta']))


def kernel(x, edge_index, params):
    p = params
    row = lambda v: v.reshape(1, -1)
    loops_idx = jnp.arange(N, dtype=jnp.int32)
    npad_e = EPAD - E_REAL
    src = jnp.concatenate([edge_index[0].astype(jnp.int32), loops_idx,
                           jnp.zeros((npad_e,), jnp.int32)])
    pad_dst = N + jnp.arange(npad_e, dtype=jnp.int32) % (NPAD - N)
    dst = jnp.concatenate([edge_index[1].astype(jnp.int32), loops_idx, pad_dst])
    src3 = src.reshape(NW, NCH, C)
    dst3 = dst.reshape(NW, NCH, C)
    zrow = jnp.zeros((STRIPE, F), jnp.float32)
    zden = jnp.zeros((STRIPE,), jnp.float32)

    g0 = p['gat'][0]
    h, z, a_s, a_d, u = _head_call(
        x, p['proj_w'], row(p['proj_b']), row(p['ln_g']), row(p['ln_b']),
        g0['W'], row(g0['att_src'][0]), row(g0['att_dst'][0]))
    for i in range(N_LAYERS):
        g = p['gat'][i]
        num, den = _edge_kernel(src3, dst3,
                                a_s.reshape(NPAD), a_d.reshape(NPAD),
                                u.reshape(NPAD), z, zrow, zden)
        bias, bnw, bnb = (row(g['bias']), row(p['bn'][i]['w']),
                          row(p['bn'][i]['b']))
        if i < N_LAYERS - 1:
            gn = p['gat'][i + 1]
            h, z, a_s, a_d, u = _mid_call(
                num, den, bias, bnw, bnb, h,
                gn['W'], row(gn['att_src'][0]), row(gn['att_dst'][0]))
        else:
            return _pool_call(num, den, bias, bnw, bnb, h, p)


# speedup vs baseline: 2.6274x; 1.0464x over previous
import functools

import jax
import jax.numpy as jnp
from jax import lax
from jax.experimental import pallas as pl
from jax.experimental.pallas import tpu as pltpu
from jax.experimental.pallas import tpu_sc as plsc

N = 10000
NPAD = 10112
NDEN = 10240
DSTRIPE = NDEN // 16
D_IN = 256
F = 128
N_LAYERS = 3
E_REAL = 170000
NW = 32
C = 64
NCH = 84
NBUF = 3
NCHT = NW * NCH
EPAD = NCHT * C
STRIPE = NPAD // 16


def _elu(x):
    return jnp.where(x > 0, x, jnp.exp(x) - 1.0)


def _leaky(x):
    return jnp.maximum(x, 0.2 * x)


def _att_out(z, asv, adv, a_ref, d_ref, am_ref):
    a_s = jnp.sum(z * asv, axis=1, keepdims=True)
    a_d = jnp.sum(z * adv, axis=1, keepdims=True)
    amax = jnp.max(a_s)
    pad = jnp.zeros((NPAD - N, 1), jnp.float32)
    a_ref[pl.ds(0, N), :] = a_s
    a_ref[pl.ds(N, NPAD - N), :] = pad
    d_ref[pl.ds(0, N), :] = a_d
    d_ref[pl.ds(N, NPAD - N), :] = pad
    am_ref[...] = jnp.zeros((1, F), jnp.float32) + amax


def _head_body(x_ref, w_ref, b_ref, g_ref, beta_ref, gw_ref, asv_ref, adv_ref,
               h_ref, z_ref, a_ref, d_ref, am_ref):
    h = jnp.dot(x_ref[...], w_ref[...].T, preferred_element_type=jnp.float32)
    h = h + b_ref[...]
    mu = jnp.mean(h, axis=1, keepdims=True)
    var = jnp.mean((h - mu) ** 2, axis=1, keepdims=True)
    h = (h - mu) / jnp.sqrt(var + 1e-5) * g_ref[...] + beta_ref[...]
    h = _elu(h)
    h_ref[...] = h
    z = jnp.dot(h, gw_ref[...].T, preferred_element_type=jnp.float32)
    z_ref[...] = z
    _att_out(z, asv_ref[...], adv_ref[...], a_ref, d_ref, am_ref)


def _head_call(x, w, b, g, beta, gw, asv, adv):
    return pl.pallas_call(
        _head_body,
        out_shape=(
            jax.ShapeDtypeStruct((N, F), jnp.float32),
            jax.ShapeDtypeStruct((N, F), jnp.float32),
            jax.ShapeDtypeStruct((NPAD, 1), jnp.float32),
            jax.ShapeDtypeStruct((NPAD, 1), jnp.float32),
            jax.ShapeDtypeStruct((1, F), jnp.float32),
        ),
    )(x, w, b, g, beta, gw, asv, adv)


def _post_h(num_ref, den_ref, bias_ref, bnw_ref, bnb_ref, hin_ref):
    n = num_ref[pl.ds(0, N), :] + num_ref[pl.ds(NPAD, N), :]
    d = den_ref[pl.ds(0, N), :] + den_ref[pl.ds(NDEN, N), :]
    out = n / (d + 1e-16) + bias_ref[...]
    out = out / jnp.sqrt(1.0 + 1e-5) * bnw_ref[...] + bnb_ref[...]
    return _elu(out) + hin_ref[...]


def _mid_body(num_ref, den_ref, bias_ref, bnw_ref, bnb_ref, hin_ref,
              gw_ref, asv_ref, adv_ref,
              h_ref, z_ref, a_ref, d_ref, am_ref):
    h = _post_h(num_ref, den_ref, bias_ref, bnw_ref, bnb_ref, hin_ref)
    h_ref[...] = h
    z = jnp.dot(h, gw_ref[...].T, preferred_element_type=jnp.float32)
    z_ref[...] = z
    _att_out(z, asv_ref[...], adv_ref[...], a_ref, d_ref, am_ref)


def _mid_call(num, den, bias, bnw, bnb, hin, gw, asv, adv):
    return pl.pallas_call(
        _mid_body,
        out_shape=(
            jax.ShapeDtypeStruct((N, F), jnp.float32),
            jax.ShapeDtypeStruct((N, F), jnp.float32),
            jax.ShapeDtypeStruct((NPAD, 1), jnp.float32),
            jax.ShapeDtypeStruct((NPAD, 1), jnp.float32),
            jax.ShapeDtypeStruct((1, F), jnp.float32),
        ),
    )(num, den.reshape(2 * NDEN, 1), bias, bnw, bnb, hin, gw, asv, adv)


_SC_MESH = plsc.VectorSubcoreMesh(core_axis_name="c", subcore_axis_name="s")


@functools.partial(
    pl.kernel,
    out_type=(
        jax.ShapeDtypeStruct((2 * NPAD, F), jnp.float32),
        jax.ShapeDtypeStruct((2 * NDEN,), jnp.float32),
    ),
    mesh=_SC_MESH,
    compiler_params=pltpu.CompilerParams(needs_layout_passes=False),
    scratch_types=[
        pltpu.VMEM((NCH, C), jnp.int32),
        pltpu.VMEM((NCH, C), jnp.int32),
        pltpu.VMEM((NBUF, C), jnp.float32),
        pltpu.VMEM((NBUF, C), jnp.float32),
        pltpu.VMEM((NBUF, C), jnp.float32),
        pltpu.VMEM((NBUF, C, F), jnp.float32),
        pltpu.VMEM((F,), jnp.float32),
        pltpu.VMEM_SHARED((NPAD, F), jnp.float32),
        pltpu.VMEM_SHARED((NDEN,), jnp.float32),
        pltpu.SemaphoreType.DMA,
        pltpu.SemaphoreType.DMA,
    ],
)
def _edge_kernel(src_hbm, dst_hbm, asrc_hbm, adst_hbm, am_hbm, z_hbm,
                 zrow_hbm, zden_hbm, num_hbm, den_hbm,
                 src_v, dst_v, as_v, ad_v, w_v, rows_v, av_v,
                 num_sh, den_sh, sem_g, sem_s):
    c = lax.axis_index("c")
    s = lax.axis_index("s")
    wid = c * 16 + s

    pltpu.sync_copy(zrow_hbm, num_sh.at[pl.ds(s * STRIPE, STRIPE)])
    pltpu.sync_copy(zden_hbm, den_sh.at[pl.ds(s * DSTRIPE, DSTRIPE)])
    pltpu.sync_copy(src_hbm.at[wid], src_v)
    pltpu.sync_copy(dst_hbm.at[wid], dst_v)
    pltpu.sync_copy(am_hbm, av_v)
    plsc.subcore_barrier()

    def issue_gather(j, b):
        pltpu.async_copy(z_hbm.at[src_v.at[j]], rows_v.at[b], sem_g)
        pltpu.async_copy(asrc_hbm.at[src_v.at[j]], as_v.at[b], sem_g)
        pltpu.async_copy(adst_hbm.at[dst_v.at[j]], ad_v.at[b], sem_g)

    def wait_gather(j, b):
        pltpu.make_async_copy(z_hbm.at[src_v.at[j]], rows_v.at[b], sem_g).wait()
        pltpu.make_async_copy(asrc_hbm.at[src_v.at[j]], as_v.at[b], sem_g).wait()
        pltpu.make_async_copy(adst_hbm.at[dst_v.at[j]], ad_v.at[b], sem_g).wait()

    def issue_scatter(j, b):
        pltpu.async_copy(w_v.at[b], den_sh.at[dst_v.at[j]], sem_s, add=True)
        pltpu.async_copy(rows_v.at[b], num_sh.at[dst_v.at[j]], sem_s, add=True)

    def wait_scatter(j, b):
        pltpu.make_async_copy(w_v.at[b], den_sh.at[dst_v.at[j]], sem_s).wait()
        pltpu.make_async_copy(rows_v.at[b], num_sh.at[dst_v.at[j]], sem_s).wait()

    def compute(b):
        amax = av_v[pl.ds(0, 16)]
        for g in range(C // 16):
            k = g * 16
            a_d = ad_v[b, pl.ds(k, 16)]
            t = _leaky(as_v[b, pl.ds(k, 16)] + a_d)
            w_v[b, pl.ds(k, 16)] = jnp.exp(t - _leaky(amax + a_d))

        @plsc.parallel_loop(0, C, unroll=4)
        def _scale(e):
            wsp = plsc.load_gather(
                w_v, [jnp.full((16,), b, jnp.int32), jnp.full((16,), e, jnp.int32)])
            for kk in range(F // 16):
                rows_v[b, e, pl.ds(kk * 16, 16)] = (
                    rows_v[b, e, pl.ds(kk * 16, 16)] * wsp)

    def sub_block(j, q_off, b):
        q = j + q_off
        wait_gather(q, b)
        compute(b)
        issue_scatter(q, b)
        bn = (b + 3) % NBUF

        @pl.when(q + 3 < NCH)
        def _():
            @pl.when(q > 0)
            def _():
                wait_scatter(q - 1, bn)
            issue_gather(q + 3, bn)

    issue_gather(0, 0)
    issue_gather(1, 1)
    issue_gather(2, 2)

    @pl.loop(0, NCH, step=NBUF)
    def _row_loop(j):
        for b in range(NBUF):
            sub_block(j, b, b)

    for i in range(NBUF):
        q = NCH - NBUF + i
        wait_scatter(q, q % NBUF)

    plsc.subcore_barrier()
    pltpu.sync_copy(num_sh.at[pl.ds(s * STRIPE, STRIPE)],
                    num_hbm.at[pl.ds(c * NPAD + s * STRIPE, STRIPE)])
    pltpu.sync_copy(den_sh.at[pl.ds(s * DSTRIPE, DSTRIPE)],
                    den_hbm.at[pl.ds(c * NDEN + s * DSTRIPE, DSTRIPE)])


def _pool_body(num_ref, den_ref, bias_ref, bnw_ref, bnb_ref, hin_ref,
               mw_ref, mb_ref, xw_ref, xb_ref, a1w_ref, a1b_ref,
               a2w_ref, a2b_ref, f1w_ref, f1b_ref, f2w_ref, f2b_ref,
               fg_ref, fbeta_ref, out_ref):
    h = _post_h(num_ref, den_ref, bias_ref, bnw_ref, bnb_ref, hin_ref)
    mean_h = jnp.mean(h, axis=0, keepdims=True)
    mp = jnp.dot(mean_h, mw_ref[...].T, preferred_element_type=jnp.float32)
    mp = mp + mb_ref[...]
    max_h = jnp.max(h, axis=0, keepdims=True)
    xp = jnp.dot(max_h, xw_ref[...].T, preferred_element_type=jnp.float32)
    xp = xp + xb_ref[...]
    s1 = jnp.dot(h, a1w_ref[...].T, preferred_element_type=jnp.float32)
    s1 = jnp.maximum(s1 + a1b_ref[...], 0.0)
    s = jnp.sum(s1 * a2w_ref[...], axis=1, keepdims=True)
    s = s + a2b_ref[0, 0]
    smax = jnp.max(s)
    es = jnp.exp(s - smax)
    sw = es / jnp.sum(es)
    ah = jnp.sum(h * sw, axis=0, keepdims=True)
    ap = jnp.dot(ah, mw_ref[...].T, preferred_element_type=jnp.float32)
    comb = jnp.concatenate([mp, xp, ap], axis=1)
    f = jnp.dot(comb, f1w_ref[...].T, preferred_element_type=jnp.float32)
    f = jnp.maximum(f + f1b_ref[...], 0.0)
    f = jnp.dot(f, f2w_ref[...].T, preferred_element_type=jnp.float32)
    f = f + f2b_ref[...]
    mu = jnp.mean(f, axis=1, keepdims=True)
    var = jnp.mean((f - mu) ** 2, axis=1, keepdims=True)
    out_ref[...] = (f - mu) / jnp.sqrt(var + 1e-5) * fg_ref[...] + fbeta_ref[...]


def _pool_call(num, den, bias, bnw, bnb, hin, p):
    row = lambda v: v.reshape(1, -1)
    return pl.pallas_call(
        _pool_body,
        out_shape=jax.ShapeDtypeStruct((1, F), jnp.float32),
    )(num, den.reshape(2 * NDEN, 1), bias, bnw, bnb, hin,
      p['mean_w'], row(p['mean_b']), p['max_w'], row(p['max_b']),
      p['attn_w1'], row(p['attn_b1']), p['attn_w2'], row(p['attn_b2']),
      p['fus_w1'], row(p['fus_b1']), p['fus_w2'], row(p['fus_b2']),
      row(p['fus_g']), row(p['fus_beta']))


def kernel(x, edge_index, params):
    p = params
    row = lambda v: v.reshape(1, -1)
    loops_idx = jnp.arange(N, dtype=jnp.int32)
    npad_e = EPAD - E_REAL
    src = jnp.concatenate([edge_index[0].astype(jnp.int32), loops_idx,
                           jnp.zeros((npad_e,), jnp.int32)])
    pad_dst = N + jnp.arange(npad_e, dtype=jnp.int32) % (NPAD - N)
    dst = jnp.concatenate([edge_index[1].astype(jnp.int32), loops_idx, pad_dst])
    src3 = src.reshape(NW, NCH, C)
    dst3 = dst.reshape(NW, NCH, C)
    zrow = jnp.zeros((STRIPE, F), jnp.float32)
    zden = jnp.zeros((DSTRIPE,), jnp.float32)

    g0 = p['gat'][0]
    h, z, a_s, a_d, a_m = _head_call(
        x, p['proj_w'], row(p['proj_b']), row(p['ln_g']), row(p['ln_b']),
        g0['W'], row(g0['att_src'][0]), row(g0['att_dst'][0]))
    for i in range(N_LAYERS):
        g = p['gat'][i]
        num, den = _edge_kernel(src3, dst3,
                                a_s.reshape(NPAD), a_d.reshape(NPAD),
                                a_m.reshape(F), z, zrow, zden)
        bias, bnw, bnb = (row(g['bias']), row(p['bn'][i]['w']),
                          row(p['bn'][i]['b']))
        if i < N_LAYERS - 1:
            gn = p['gat'][i + 1]
            h, z, a_s, a_d, a_m = _mid_call(
                num, den, bias, bnw, bnb, h,
                gn['W'], row(gn['att_src'][0]), row(gn['att_dst'][0]))
        else:
            return _pool_call(num, den, bias, bnw, bnb, h, p)
